# baseline (speedup 1.0000x reference)
import functools
import jax
import jax.numpy as jnp
from jax import lax
from jax.experimental import pallas as pl
from jax.experimental.pallas import tpu as pltpu
from jax.experimental.pallas import tpu_sc as plsc

MAXREL = 14
NT = 2 * MAXREL + 2
NU = 64
LQ = 577
S = 24
NW = 32
NV = NU // 16


def _sc_body(tv_hbm, th_hbm, res_hbm, out_hbm,
             tv_v, th_v, res_v, hrow_v, buf_v):
    cid = lax.axis_index("c")
    sid = lax.axis_index("s")
    w = sid * 2 + cid

    pltpu.sync_copy(tv_hbm, tv_v)
    pltpu.sync_copy(th_hbm, th_v)
    pltpu.sync_copy(res_hbm, res_v)
    res = res_v[pl.ds(0, 16)]

    def vloop(v, carry):
        for u in range(NV):
            tv_v[v, pl.ds(16 * u, 16)] = tv_v[v, pl.ds(16 * u, 16)] + res
        return carry
    lax.fori_loop(0, NT, vloop, 0)

    @pl.when(w == 0)
    def _row0():
        def jloop(j, carry):
            for u in range(NV):
                buf_v[j, pl.ds(16 * u, 16)] = (tv_v[0, pl.ds(16 * u, 16)] +
                                               th_v[0, pl.ds(16 * u, 16)])
            return carry
        lax.fori_loop(0, LQ, jloop, 0)
        pltpu.sync_copy(buf_v, out_hbm.at[0])

    def krow(k, carry):
        i = 1 + w + NW * k
        im = i - 1
        a = im // S
        r = im - a * S
        for t in range(S):
            ch = jnp.maximum(jnp.minimum(t - r, MAXREL), -MAXREL) + MAXREL + 1
            for u in range(NV):
                hrow_v[t, pl.ds(16 * u, 16)] = th_v[ch, pl.ds(16 * u, 16)]
        for u in range(NV):
            buf_v[0, pl.ds(16 * u, 16)] = (tv_v[0, pl.ds(16 * u, 16)] +
                                           th_v[0, pl.ds(16 * u, 16)])

        def bloop(b, carry):
            d = jnp.maximum(jnp.minimum(b - a, MAXREL), -MAXREL)
            cv = d + MAXREL + 1
            base = 1 + S * b
            tvr = [tv_v[cv, pl.ds(16 * u, 16)] for u in range(NV)]
            for t in range(S):
                for u in range(NV):
                    buf_v[base + t, pl.ds(16 * u, 16)] = \
                        tvr[u] + hrow_v[t, pl.ds(16 * u, 16)]
            return carry
        lax.fori_loop(0, S, bloop, 0)
        pltpu.sync_copy(buf_v, out_hbm.at[i])
        return carry
    lax.fori_loop(0, (LQ - 1) // NW, krow, 0)


def kernel(table_v, table_h, length_q, length_k):
    resf = jnp.asarray((length_q - 577) + (length_k - 577), jnp.float32)
    res_arr = jnp.full((16,), resf, jnp.float32)
    mesh = plsc.VectorSubcoreMesh(core_axis_name="c", subcore_axis_name="s")
    k = functools.partial(
        pl.kernel,
        mesh=mesh,
        out_type=jax.ShapeDtypeStruct((LQ, LQ, NU), jnp.float32),
        scratch_types=[
            pltpu.VMEM((NT, NU), jnp.float32),
            pltpu.VMEM((NT, NU), jnp.float32),
            pltpu.VMEM((16,), jnp.float32),
            pltpu.VMEM((S, NU), jnp.float32),
            pltpu.VMEM((LQ, NU), jnp.float32),
        ],
    )(_sc_body)
    return k(table_v, table_h, res_arr)


# speedup vs baseline: 1.5542x; 1.5542x over previous
import functools
import jax
import jax.numpy as jnp
from jax import lax
from jax.experimental import pallas as pl
from jax.experimental.pallas import tpu as pltpu
from jax.experimental.pallas import tpu_sc as plsc

MAXREL = 14
NT = 2 * MAXREL + 2
NU = 64
LQ = 577
S = 24
NW = 32
NV = NU // 16
HA = 288
HB = LQ - HA


def _clip(x):
    return jnp.maximum(jnp.minimum(x, MAXREL), -MAXREL) + MAXREL + 1


def _sc_body(tv_hbm, th_hbm, res_hbm, out_hbm,
             tv_v, th_v, res_v, hrow_v, bufa_v, bufb_v, sema, semb):
    cid = lax.axis_index("c")
    sid = lax.axis_index("s")
    w = sid * 2 + cid

    pltpu.sync_copy(tv_hbm, tv_v)
    pltpu.sync_copy(th_hbm, th_v)
    pltpu.sync_copy(res_hbm, res_v)
    res = res_v[pl.ds(0, 16)]

    def vloop(v, carry):
        for u in range(NV):
            tv_v[v, pl.ds(16 * u, 16)] = tv_v[v, pl.ds(16 * u, 16)] + res
        return carry
    lax.fori_loop(0, NT, vloop, 0)

    t0 = [tv_v[0, pl.ds(16 * u, 16)] + th_v[0, pl.ds(16 * u, 16)]
          for u in range(NV)]

    @pl.when(w == 0)
    def _row0():
        def jla(j, carry):
            for u in range(NV):
                bufa_v[j, pl.ds(16 * u, 16)] = t0[u]
            return carry
        lax.fori_loop(0, HA, jla, 0)
        pltpu.sync_copy(bufa_v, out_hbm.at[0, pl.ds(0, HA)])

        def jlb(j, carry):
            for u in range(NV):
                bufb_v[j, pl.ds(16 * u, 16)] = t0[u]
            return carry
        lax.fori_loop(0, HB, jlb, 0)
        pltpu.sync_copy(bufb_v, out_hbm.at[0, pl.ds(HA, HB)])

    def vrow(cv):
        return [tv_v[cv, pl.ds(16 * u, 16)] for u in range(NV)]

    def put(buf, row, tvr, t):
        for u in range(NV):
            buf[row, pl.ds(16 * u, 16)] = tvr[u] + hrow_v[t, pl.ds(16 * u, 16)]

    def fill_a(a, r):
        for u in range(NV):
            bufa_v[0, pl.ds(16 * u, 16)] = t0[u]

        @plsc.parallel_loop(0, 11, unroll=2)
        def _(b):
            tvr = vrow(_clip(b - a))
            for t in range(S):
                put(bufa_v, 1 + S * b + t, tvr, t)
        tvr = vrow(_clip(11 - a))
        for t in range(23):
            put(bufa_v, 1 + S * 11 + t, tvr, t)

    def fill_b(a, r):
        tvr = vrow(_clip(11 - a))
        put(bufb_v, 0, tvr, 23)

        @plsc.parallel_loop(12, S, unroll=2)
        def _(b):
            tvr = vrow(_clip(b - a))
            for t in range(S):
                put(bufb_v, 1 + S * b + t - HA, tvr, t)

    def krow(k, carry):
        i = 1 + w + NW * k
        im = i - 1
        a = im // S
        r = im - a * S
        for t in range(S):
            ch = _clip(t - r)
            for u in range(NV):
                hrow_v[t, pl.ds(16 * u, 16)] = th_v[ch, pl.ds(16 * u, 16)]

        @pl.when(k > 0)
        def _():
            pltpu.make_async_copy(
                out_hbm.at[1, pl.ds(0, HA)], bufa_v, sema).wait()
        fill_a(a, r)
        pltpu.async_copy(bufa_v, out_hbm.at[i, pl.ds(0, HA)], sema)

        @pl.when(k > 0)
        def _():
            pltpu.make_async_copy(
                out_hbm.at[1, pl.ds(HA, HB)], bufb_v, semb).wait()
        fill_b(a, r)
        pltpu.async_copy(bufb_v, out_hbm.at[i, pl.ds(HA, HB)], semb)
        return carry
    lax.fori_loop(0, (LQ - 1) // NW, krow, 0)

    pltpu.make_async_copy(out_hbm.at[1, pl.ds(0, HA)], bufa_v, sema).wait()
    pltpu.make_async_copy(out_hbm.at[1, pl.ds(HA, HB)], bufb_v, semb).wait()


def kernel(table_v, table_h, length_q, length_k):
    resf = jnp.asarray((length_q - 577) + (length_k - 577), jnp.float32)
    res_arr = jnp.full((16,), resf, jnp.float32)
    mesh = plsc.VectorSubcoreMesh(core_axis_name="c", subcore_axis_name="s")
    k = functools.partial(
        pl.kernel,
        mesh=mesh,
        out_type=jax.ShapeDtypeStruct((LQ, LQ, NU), jnp.float32),
        scratch_types=[
            pltpu.VMEM((NT, NU), jnp.float32),
            pltpu.VMEM((NT, NU), jnp.float32),
            pltpu.VMEM((16,), jnp.float32),
            pltpu.VMEM((S, NU), jnp.float32),
            pltpu.VMEM((HA, NU), jnp.float32),
            pltpu.VMEM((HB, NU), jnp.float32),
            pltpu.SemaphoreType.DMA,
            pltpu.SemaphoreType.DMA,
        ],
    )(_sc_body)
    return k(table_v, table_h, res_arr)


# speedup vs baseline: 2.2276x; 1.4333x over previous
import jax
import jax.numpy as jnp
from jax import lax
from jax.experimental import pallas as pl
from jax.experimental.pallas import tpu as pltpu

MAXREL = 14
NT = 2 * MAXREL + 2
NU = 64
LQ = 577
S = 24


def _body(tv_ref, th_ref, res_ref, out_ref, rowv_ref, rowhr_ref, t0_ref):
    g = pl.program_id(0)

    @pl.when(g == 0)
    def _precompute():
        tv = tv_ref[:, :]
        th = th_ref[:, :]
        res = res_ref[0]
        p = lax.broadcasted_iota(jnp.int32, (S * S, NT), 0)
        l = lax.broadcasted_iota(jnp.int32, (S * S, NT), 1)
        idx = jnp.clip(p % S - p // S, -MAXREL, MAXREL) + MAXREL + 1
        oh = (l == idx).astype(jnp.float32)
        vflat = jnp.dot(oh, tv, preferred_element_type=jnp.float32) + res
        hflat = jnp.dot(oh, th, preferred_element_type=jnp.float32)
        tv0 = tv[0:1, :] + res
        th0 = th[0:1, :]
        for a in range(S):
            blk = vflat[S * a:S * (a + 1)]
            rep = jnp.broadcast_to(blk[:, None, :], (S, S, NU))
            rowv_ref[a, 0:1, :] = tv0
            rowv_ref[a, 1:LQ, :] = rep.reshape(S * S, NU)
        rowv_ref[S, :, :] = jnp.broadcast_to(tv0, (LQ, NU))
        for r in range(S):
            blk = hflat[S * r:S * (r + 1)]
            til = jnp.broadcast_to(blk[None, :, :], (S, S, NU))
            k = (r + 1) % S
            rowhr_ref[k, 0:1, :] = th0
            rowhr_ref[k, 1:LQ, :] = til.reshape(S * S, NU)
        t0_ref[0, :, :] = jnp.broadcast_to(tv0 + th0, (LQ, NU))

    vg = rowv_ref[pl.ds(g, 1), :, :]
    out_ref[:, :, :] = (jnp.broadcast_to(vg, (S, LQ, NU)) +
                        rowhr_ref[:, :, :])

    @pl.when(g == 0)
    def _():
        out_ref[0:1, :, :] = t0_ref[:, :, :]

    @pl.when(g > 0)
    def _():
        gm = jnp.maximum(g - 1, 0)
        out_ref[0:1, :, :] = (rowv_ref[pl.ds(gm, 1), :, :] +
                              rowhr_ref[0:1, :, :])


def kernel(table_v, table_h, length_q, length_k):
    res = jnp.asarray((length_q - 577) + (length_k - 577),
                      jnp.float32).reshape(1)
    out = pl.pallas_call(
        _body,
        grid=(S + 1,),
        in_specs=[
            pl.BlockSpec((NT, NU), lambda g: (0, 0)),
            pl.BlockSpec((NT, NU), lambda g: (0, 0)),
            pl.BlockSpec(memory_space=pltpu.SMEM),
        ],
        out_specs=pl.BlockSpec((S, LQ, NU), lambda g: (g, 0, 0)),
        out_shape=jax.ShapeDtypeStruct((LQ, LQ, NU), jnp.float32),
        scratch_shapes=[
            pltpu.VMEM((S + 1, LQ, NU), jnp.float32),
            pltpu.VMEM((S, LQ, NU), jnp.float32),
            pltpu.VMEM((1, LQ, NU), jnp.float32),
        ],
    )(table_v, table_h, res)
    return out
